# baseline (speedup 1.0000x reference)
import functools

import jax
import jax.numpy as jnp
from jax import lax
from jax.experimental import pallas as pl
from jax.experimental.pallas import tpu as pltpu
from jax.experimental.pallas import tpu_sc as plsc

NC = 2
NS = 16
NW = NC * NS
LANES = 16


@functools.lru_cache(maxsize=None)
def _build(B, D, K, POOL, T_CHUNK):
    TOK_PER_W = B // NW
    N_CHUNK = TOK_PER_W // T_CHUNK
    R = T_CHUNK * K

    mesh = plsc.VectorSubcoreMesh(core_axis_name="c", subcore_axis_name="s")

    @functools.partial(
        pl.kernel,
        out_type=jax.ShapeDtypeStruct((B, D), jnp.float32),
        mesh=mesh,
        scratch_types=[
            pltpu.VMEM((TOK_PER_W * K,), jnp.int32),
            pltpu.VMEM((2, R), jnp.float32),
            pltpu.VMEM((2, T_CHUNK, D), jnp.float32),
            pltpu.VMEM((2, R, D), jnp.float32),
            pltpu.VMEM((R, D // 2), jnp.uint32),
            pltpu.VMEM((2, T_CHUNK, D), jnp.float32),
            pltpu.SemaphoreType.DMA((2,)),
            pltpu.SemaphoreType.DMA((2,)),
            pltpu.SemaphoreType.DMA((2,)),
            pltpu.SemaphoreType.DMA((2,)),
        ],
    )
    def sc_kernel(x_hbm, idx_hbm, w_hbm, pool_hbm, out_hbm,
                  idx_v, w_v, x_v, rows_v, pk_v, out_v,
                  sem_g, sem_x, sem_w, sem_o):
        wid = lax.axis_index("s") * NC + lax.axis_index("c")
        tok0 = wid * TOK_PER_W

        pltpu.sync_copy(idx_hbm.at[pl.ds(tok0 * K, TOK_PER_W * K)], idx_v)

        def issue_rows(ci, b):
            pltpu.async_copy(pool_hbm.at[idx_v.at[pl.ds(ci * R, R)]],
                             rows_v.at[b], sem_g.at[b])

        def issue_xw(ci, b):
            base_t = tok0 + ci * T_CHUNK
            pltpu.async_copy(x_hbm.at[pl.ds(base_t, T_CHUNK)],
                             x_v.at[b], sem_x.at[b])
            pltpu.async_copy(w_hbm.at[pl.ds(base_t * K, R)],
                             w_v.at[b], sem_w.at[b])

        issue_rows(0, 0)
        issue_xw(0, 0)

        def chunk_body(ci, _):
            buf = lax.rem(ci, 2)
            nb = 1 - buf
            base_t = tok0 + ci * T_CHUNK

            @pl.when(ci + 1 < N_CHUNK)
            def _():
                issue_rows(ci + 1, nb)
                issue_xw(ci + 1, nb)

            pltpu.make_async_copy(pool_hbm.at[idx_v.at[pl.ds(ci * R, R)]],
                                  rows_v.at[buf], sem_g.at[buf]).wait()
            pltpu.make_async_copy(x_hbm.at[pl.ds(base_t, T_CHUNK)],
                                  x_v.at[buf], sem_x.at[buf]).wait()
            pltpu.make_async_copy(w_hbm.at[pl.ds(base_t * K, R)],
                                  w_v.at[buf], sem_w.at[buf]).wait()

            @pl.when(ci >= 2)
            def _():
                pltpu.make_async_copy(
                    out_v.at[buf], out_hbm.at[pl.ds(base_t, T_CHUNK)],
                    sem_o.at[buf]).wait()

            rows_b = rows_v.at[buf]
            x_b = x_v.at[buf]
            w_b = w_v.at[buf]
            out_b = out_v.at[buf]

            def tok_body(t, _):
                row0 = t * K

                m_hi = jnp.uint32(0xFFFF0000)

                def dot_body(c, accs):
                    x_lo = x_b[t, pl.ds(c, LANES)]
                    x_hi = x_b[t, pl.ds(c + LANES, LANES)]
                    ph = pl.multiple_of(lax.shift_right_logical(c, 1), LANES)
                    new = []
                    for k in range(K):
                        lo = rows_b[row0 + k, pl.ds(c, LANES)]
                        hi = rows_b[row0 + k, pl.ds(c + LANES, LANES)]
                        u = ((lax.bitcast_convert_type(hi, jnp.uint32) & m_hi)
                             | (lax.bitcast_convert_type(lo, jnp.uint32) >> 16))
                        pk_v[row0 + k, pl.ds(ph, LANES)] = u
                        new.append(accs[k] + x_lo * lo + x_hi * hi)
                    return tuple(new)

                accs = plsc.parallel_loop(
                    0, D, 2 * LANES, unroll=4,
                    carry=tuple(jnp.zeros((LANES,), jnp.float32)
                                for _ in range(K)))(dot_body)

                lane = lax.iota(jnp.int32, LANES)
                wvec = w_b[pl.ds(lax.div(t, 2) * LANES, LANES)]
                woff = lax.rem(t, 2) * K
                coefs = []
                for k in range(K):
                    v = accs[k]
                    for s in (8, 4, 2, 1):
                        v = v + v.at[lane ^ s].get(mode="promise_in_bounds")
                    e = jnp.exp(2.0 * v)
                    tanh_v = 1.0 - 2.0 / (e + 1.0)
                    wk = wvec.at[jnp.full((LANES,), woff + k, jnp.int32)].get(
                        mode="promise_in_bounds")
                    coefs.append(tanh_v * wk)

                def comb_body(c):
                    ph = pl.multiple_of(lax.shift_right_logical(c, 1), LANES)
                    acc_lo = x_b[t, pl.ds(c, LANES)]
                    acc_hi = x_b[t, pl.ds(c + LANES, LANES)]
                    for k in range(K):
                        u = pk_v[row0 + k, pl.ds(ph, LANES)]
                        lo_f = lax.bitcast_convert_type(u << 16, jnp.float32)
                        hi_f = lax.bitcast_convert_type(u & m_hi, jnp.float32)
                        acc_lo = acc_lo + coefs[k] * lo_f
                        acc_hi = acc_hi + coefs[k] * hi_f
                    out_b[t, pl.ds(c, LANES)] = acc_lo
                    out_b[t, pl.ds(c + LANES, LANES)] = acc_hi

                plsc.parallel_loop(0, D, 2 * LANES, unroll=4)(comb_body)
                return 0

            lax.fori_loop(0, T_CHUNK, tok_body, 0)
            pltpu.async_copy(out_b, out_hbm.at[pl.ds(base_t, T_CHUNK)],
                             sem_o.at[buf])
            return 0

        lax.fori_loop(0, N_CHUNK, chunk_body, 0)

        for b in range(2):
            ci = N_CHUNK - 2 + b
            base_t = tok0 + ci * T_CHUNK
            pltpu.make_async_copy(
                out_v.at[ci % 2], out_hbm.at[pl.ds(base_t, T_CHUNK)],
                sem_o.at[ci % 2]).wait()

    return sc_kernel


def kernel(x, indices, weights, pool):
    B, D = x.shape
    K = indices.shape[1]
    idx = indices.astype(jnp.int32).reshape(-1)
    w = weights.astype(jnp.float32).reshape(-1)
    return _build(B, D, K, pool.shape[0], 4)(x, idx, w, pool)


# speedup vs baseline: 1.0026x; 1.0026x over previous
import functools

import jax
import jax.numpy as jnp
from jax import lax
from jax.experimental import pallas as pl
from jax.experimental.pallas import tpu as pltpu
from jax.experimental.pallas import tpu_sc as plsc

NC = 2
NS = 16
NW = NC * NS
LANES = 16


@functools.lru_cache(maxsize=None)
def _build(B, D, K, POOL, T_CHUNK):
    TOK_PER_W = B // NW
    N_CHUNK = TOK_PER_W // T_CHUNK
    R = T_CHUNK * K

    mesh = plsc.VectorSubcoreMesh(core_axis_name="c", subcore_axis_name="s")

    @functools.partial(
        pl.kernel,
        out_type=jax.ShapeDtypeStruct((B, D), jnp.float32),
        mesh=mesh,
        scratch_types=[
            pltpu.VMEM((TOK_PER_W * K,), jnp.int32),
            pltpu.VMEM((2, R), jnp.float32),
            pltpu.VMEM((2, T_CHUNK, D), jnp.float32),
            pltpu.VMEM((2, R, D), jnp.float32),
            pltpu.VMEM((R, D // 2), jnp.uint32),
            pltpu.VMEM((2, T_CHUNK, D), jnp.float32),
            pltpu.SemaphoreType.DMA((2,)),
            pltpu.SemaphoreType.DMA((2,)),
            pltpu.SemaphoreType.DMA((2,)),
            pltpu.SemaphoreType.DMA((2,)),
        ],
    )
    def sc_kernel(x_hbm, idx_hbm, w_hbm, pool_hbm, out_hbm,
                  idx_v, w_v, x_v, rows_v, pk_v, out_v,
                  sem_g, sem_x, sem_w, sem_o):
        wid = lax.axis_index("s") * NC + lax.axis_index("c")
        tok0 = wid * TOK_PER_W

        pltpu.sync_copy(idx_hbm.at[pl.ds(tok0 * K, TOK_PER_W * K)], idx_v)

        def issue_rows(ci, b):
            pltpu.async_copy(pool_hbm.at[idx_v.at[pl.ds(ci * R, R)]],
                             rows_v.at[b], sem_g.at[b])

        def issue_xw(ci, b):
            base_t = tok0 + ci * T_CHUNK
            pltpu.async_copy(x_hbm.at[pl.ds(base_t, T_CHUNK)],
                             x_v.at[b], sem_x.at[b])
            pltpu.async_copy(w_hbm.at[pl.ds(base_t * K, R)],
                             w_v.at[b], sem_w.at[b])

        issue_rows(0, 0)
        issue_xw(0, 0)

        def chunk_body(ci, _):
            buf = lax.rem(ci, 2)
            nb = 1 - buf
            base_t = tok0 + ci * T_CHUNK

            @pl.when(ci + 1 < N_CHUNK)
            def _():
                issue_rows(ci + 1, nb)
                issue_xw(ci + 1, nb)

            pltpu.make_async_copy(pool_hbm.at[idx_v.at[pl.ds(ci * R, R)]],
                                  rows_v.at[buf], sem_g.at[buf]).wait()
            pltpu.make_async_copy(x_hbm.at[pl.ds(base_t, T_CHUNK)],
                                  x_v.at[buf], sem_x.at[buf]).wait()
            pltpu.make_async_copy(w_hbm.at[pl.ds(base_t * K, R)],
                                  w_v.at[buf], sem_w.at[buf]).wait()

            @pl.when(ci >= 2)
            def _():
                pltpu.make_async_copy(
                    out_v.at[buf], out_hbm.at[pl.ds(base_t, T_CHUNK)],
                    sem_o.at[buf]).wait()

            rows_b = rows_v.at[buf]
            x_b = x_v.at[buf]
            w_b = w_v.at[buf]
            out_b = out_v.at[buf]

            def tok_body(t, _):
                row0 = t * K

                m_hi = jnp.uint32(0xFFFF0000)

                def dot_body(c, accs):
                    x_lo = x_b[t, pl.ds(c, LANES)]
                    x_hi = x_b[t, pl.ds(c + LANES, LANES)]
                    ph = pl.multiple_of(lax.shift_right_logical(c, 1), LANES)
                    new = []
                    for k in range(K):
                        lo = rows_b[row0 + k, pl.ds(c, LANES)]
                        hi = rows_b[row0 + k, pl.ds(c + LANES, LANES)]
                        u = ((lax.bitcast_convert_type(hi, jnp.uint32) & m_hi)
                             | (lax.bitcast_convert_type(lo, jnp.uint32) >> 16))
                        pk_v[row0 + k, pl.ds(ph, LANES)] = u
                        new.append(accs[k] + x_lo * lo + x_hi * hi)
                    return tuple(new)

                accs = plsc.parallel_loop(
                    0, D, 2 * LANES, unroll=2,
                    carry=tuple(jnp.zeros((LANES,), jnp.float32)
                                for _ in range(K)))(dot_body)

                lane = lax.iota(jnp.int32, LANES)
                wvec = w_b[pl.ds(lax.div(t, 2) * LANES, LANES)]
                woff = lax.rem(t, 2) * K
                coefs = []
                for k in range(K):
                    v = accs[k]
                    for s in (8, 4, 2, 1):
                        v = v + v.at[lane ^ s].get(mode="promise_in_bounds")
                    e = jnp.exp(2.0 * v)
                    tanh_v = 1.0 - 2.0 / (e + 1.0)
                    wk = wvec.at[jnp.full((LANES,), woff + k, jnp.int32)].get(
                        mode="promise_in_bounds")
                    coefs.append(tanh_v * wk)

                def comb_body(c):
                    ph = pl.multiple_of(lax.shift_right_logical(c, 1), LANES)
                    acc_lo = x_b[t, pl.ds(c, LANES)]
                    acc_hi = x_b[t, pl.ds(c + LANES, LANES)]
                    for k in range(K):
                        u = pk_v[row0 + k, pl.ds(ph, LANES)]
                        lo_f = lax.bitcast_convert_type(u << 16, jnp.float32)
                        hi_f = lax.bitcast_convert_type(u & m_hi, jnp.float32)
                        acc_lo = acc_lo + coefs[k] * lo_f
                        acc_hi = acc_hi + coefs[k] * hi_f
                    out_b[t, pl.ds(c, LANES)] = acc_lo
                    out_b[t, pl.ds(c + LANES, LANES)] = acc_hi

                plsc.parallel_loop(0, D, 2 * LANES, unroll=2)(comb_body)
                return 0

            lax.fori_loop(0, T_CHUNK, tok_body, 0)
            pltpu.async_copy(out_b, out_hbm.at[pl.ds(base_t, T_CHUNK)],
                             sem_o.at[buf])
            return 0

        lax.fori_loop(0, N_CHUNK, chunk_body, 0)

        for b in range(2):
            ci = N_CHUNK - 2 + b
            base_t = tok0 + ci * T_CHUNK
            pltpu.make_async_copy(
                out_v.at[ci % 2], out_hbm.at[pl.ds(base_t, T_CHUNK)],
                sem_o.at[ci % 2]).wait()

    return sc_kernel


def kernel(x, indices, weights, pool):
    B, D = x.shape
    K = indices.shape[1]
    idx = indices.astype(jnp.int32).reshape(-1)
    w = weights.astype(jnp.float32).reshape(-1)
    return _build(B, D, K, pool.shape[0], 4)(x, idx, w, pool)


# speedup vs baseline: 2.1282x; 2.1227x over previous
import functools

import jax
import jax.numpy as jnp
from jax import lax
from jax.experimental import pallas as pl
from jax.experimental.pallas import tpu as pltpu
from jax.experimental.pallas import tpu_sc as plsc

NC = 2
NS = 16
NW = NC * NS
LANES = 16


@functools.lru_cache(maxsize=None)
def _build(B, D, K, POOL, T_CHUNK):
    TOK_PER_W = B // NW
    N_CHUNK = TOK_PER_W // T_CHUNK
    R = T_CHUNK * K

    mesh = plsc.VectorSubcoreMesh(core_axis_name="c", subcore_axis_name="s")

    @functools.partial(
        pl.kernel,
        out_type=jax.ShapeDtypeStruct((B, D), jnp.float32),
        mesh=mesh,
        scratch_types=[
            pltpu.VMEM((TOK_PER_W * K,), jnp.int32),
            pltpu.VMEM((2, R), jnp.float32),
            pltpu.VMEM((2, T_CHUNK, D), jnp.float32),
            pltpu.VMEM((2, R, D), jnp.float32),
            pltpu.VMEM((2, T_CHUNK, D), jnp.float32),
            pltpu.SemaphoreType.DMA((2,)),
            pltpu.SemaphoreType.DMA((2,)),
            pltpu.SemaphoreType.DMA((2,)),
            pltpu.SemaphoreType.DMA((2,)),
        ],
    )
    def sc_kernel(x_hbm, idx_hbm, w_hbm, pool_hbm, out_hbm,
                  idx_v, w_v, x_v, rows_v, out_v,
                  sem_g, sem_x, sem_w, sem_o):
        wid = lax.axis_index("s") * NC + lax.axis_index("c")
        tok0 = wid * TOK_PER_W

        pltpu.sync_copy(idx_hbm.at[pl.ds(tok0 * K, TOK_PER_W * K)], idx_v)

        def issue_rows(ci, b):
            pltpu.async_copy(pool_hbm.at[idx_v.at[pl.ds(ci * R, R)]],
                             rows_v.at[b], sem_g.at[b])

        def issue_xw(ci, b):
            base_t = tok0 + ci * T_CHUNK
            pltpu.async_copy(x_hbm.at[pl.ds(base_t, T_CHUNK)],
                             x_v.at[b], sem_x.at[b])
            pltpu.async_copy(w_hbm.at[pl.ds(base_t * K, R)],
                             w_v.at[b], sem_w.at[b])

        issue_rows(0, 0)
        issue_xw(0, 0)

        def chunk_body(ci, _):
            buf = lax.rem(ci, 2)
            nb = 1 - buf
            base_t = tok0 + ci * T_CHUNK

            @pl.when(ci + 1 < N_CHUNK)
            def _():
                issue_rows(ci + 1, nb)
                issue_xw(ci + 1, nb)

            pltpu.make_async_copy(pool_hbm.at[idx_v.at[pl.ds(ci * R, R)]],
                                  rows_v.at[buf], sem_g.at[buf]).wait()
            pltpu.make_async_copy(x_hbm.at[pl.ds(base_t, T_CHUNK)],
                                  x_v.at[buf], sem_x.at[buf]).wait()
            pltpu.make_async_copy(w_hbm.at[pl.ds(base_t * K, R)],
                                  w_v.at[buf], sem_w.at[buf]).wait()

            @pl.when(ci >= 2)
            def _():
                pltpu.make_async_copy(
                    out_v.at[buf], out_hbm.at[pl.ds(base_t, T_CHUNK)],
                    sem_o.at[buf]).wait()

            rows_b = rows_v.at[buf]
            x_b = x_v.at[buf]
            w_b = w_v.at[buf]
            out_b = out_v.at[buf]

            def tok_body(t, _):
                row0 = t * K

                def dot_body(c, accs):
                    xv = x_b[t, pl.ds(c, LANES)]
                    return tuple(
                        accs[k] + xv * rows_b[row0 + k, pl.ds(c, LANES)]
                        for k in range(K))

                accs = plsc.parallel_loop(
                    0, D, LANES, unroll=8,
                    carry=tuple(jnp.zeros((LANES,), jnp.float32)
                                for _ in range(K)))(dot_body)

                lane = lax.iota(jnp.int32, LANES)
                wvec = w_b[pl.ds(lax.div(t, 2) * LANES, LANES)]
                woff = lax.rem(t, 2) * K
                coefs = []
                for k in range(K):
                    v = accs[k]
                    for s in (8, 4, 2, 1):
                        v = v + v.at[lane ^ s].get(mode="promise_in_bounds")
                    e = jnp.exp(2.0 * v)
                    tanh_v = 1.0 - 2.0 / (e + 1.0)
                    wk = wvec.at[jnp.full((LANES,), woff + k, jnp.int32)].get(
                        mode="promise_in_bounds")
                    coefs.append(tanh_v * wk)

                def comb_body(c):
                    o = x_b[t, pl.ds(c, LANES)]
                    for k in range(K):
                        o = o + coefs[k] * rows_b[row0 + k, pl.ds(c, LANES)]
                    out_b[t, pl.ds(c, LANES)] = o

                plsc.parallel_loop(0, D, LANES, unroll=8)(comb_body)
                return 0

            lax.fori_loop(0, T_CHUNK, tok_body, 0)
            pltpu.async_copy(out_b, out_hbm.at[pl.ds(base_t, T_CHUNK)],
                             sem_o.at[buf])
            return 0

        lax.fori_loop(0, N_CHUNK, chunk_body, 0)

        for b in range(2):
            ci = N_CHUNK - 2 + b
            base_t = tok0 + ci * T_CHUNK
            pltpu.make_async_copy(
                out_v.at[ci % 2], out_hbm.at[pl.ds(base_t, T_CHUNK)],
                sem_o.at[ci % 2]).wait()

    return sc_kernel


def kernel(x, indices, weights, pool):
    B, D = x.shape
    K = indices.shape[1]
    idx = indices.astype(jnp.int32).reshape(-1)
    w = weights.astype(jnp.float32).reshape(-1)
    return _build(B, D, K, pool.shape[0], 4)(x, idx, w, pool)


# speedup vs baseline: 2.1475x; 1.0090x over previous
import functools

import jax
import jax.numpy as jnp
from jax import lax
from jax.experimental import pallas as pl
from jax.experimental.pallas import tpu as pltpu
from jax.experimental.pallas import tpu_sc as plsc

NC = 2
NS = 16
NW = NC * NS
LANES = 16


@functools.lru_cache(maxsize=None)
def _build(B, D, K, POOL, T_CHUNK):
    TOK_PER_W = B // NW
    N_CHUNK = TOK_PER_W // T_CHUNK
    R = T_CHUNK * K

    mesh = plsc.VectorSubcoreMesh(core_axis_name="c", subcore_axis_name="s")

    @functools.partial(
        pl.kernel,
        out_type=jax.ShapeDtypeStruct((B, D), jnp.float32),
        mesh=mesh,
        scratch_types=[
            pltpu.VMEM((TOK_PER_W * K,), jnp.int32),
            pltpu.VMEM((2, R), jnp.float32),
            pltpu.VMEM((2, T_CHUNK, D), jnp.float32),
            pltpu.VMEM((2, R, D), jnp.float32),
            pltpu.VMEM((2, T_CHUNK, D), jnp.float32),
            pltpu.SemaphoreType.DMA((2,)),
            pltpu.SemaphoreType.DMA((2,)),
            pltpu.SemaphoreType.DMA((2,)),
            pltpu.SemaphoreType.DMA((2,)),
        ],
    )
    def sc_kernel(x_hbm, idx_hbm, w_hbm, pool_hbm, out_hbm,
                  idx_v, w_v, x_v, rows_v, out_v,
                  sem_g, sem_x, sem_w, sem_o):
        wid = lax.axis_index("s") * NC + lax.axis_index("c")
        tok0 = wid * TOK_PER_W

        pltpu.sync_copy(idx_hbm.at[pl.ds(tok0 * K, TOK_PER_W * K)], idx_v)

        def issue_rows(ci, b):
            pltpu.async_copy(pool_hbm.at[idx_v.at[pl.ds(ci * R, R)]],
                             rows_v.at[b], sem_g.at[b])

        def issue_xw(ci, b):
            base_t = tok0 + ci * T_CHUNK
            pltpu.async_copy(x_hbm.at[pl.ds(base_t, T_CHUNK)],
                             x_v.at[b], sem_x.at[b])
            pltpu.async_copy(w_hbm.at[pl.ds(base_t * K, R)],
                             w_v.at[b], sem_w.at[b])

        issue_rows(0, 0)
        issue_xw(0, 0)

        def chunk_body(ci, _):
            buf = lax.rem(ci, 2)
            nb = 1 - buf
            base_t = tok0 + ci * T_CHUNK

            @pl.when(ci + 1 < N_CHUNK)
            def _():
                issue_rows(ci + 1, nb)
                issue_xw(ci + 1, nb)

            pltpu.make_async_copy(pool_hbm.at[idx_v.at[pl.ds(ci * R, R)]],
                                  rows_v.at[buf], sem_g.at[buf]).wait()
            pltpu.make_async_copy(x_hbm.at[pl.ds(base_t, T_CHUNK)],
                                  x_v.at[buf], sem_x.at[buf]).wait()
            pltpu.make_async_copy(w_hbm.at[pl.ds(base_t * K, R)],
                                  w_v.at[buf], sem_w.at[buf]).wait()

            @pl.when(ci >= 2)
            def _():
                pltpu.make_async_copy(
                    out_v.at[buf], out_hbm.at[pl.ds(base_t, T_CHUNK)],
                    sem_o.at[buf]).wait()

            rows_b = rows_v.at[buf]
            x_b = x_v.at[buf]
            w_b = w_v.at[buf]
            out_b = out_v.at[buf]

            def tok_body(t, _):
                row0 = t * K

                def dot_body(c, accs):
                    xv = x_b[t, pl.ds(c, LANES)]
                    return tuple(
                        accs[k] + xv * rows_b[row0 + k, pl.ds(c, LANES)]
                        for k in range(K))

                accs = plsc.parallel_loop(
                    0, D, LANES, unroll=16,
                    carry=tuple(jnp.zeros((LANES,), jnp.float32)
                                for _ in range(K)))(dot_body)

                lane = lax.iota(jnp.int32, LANES)
                wvec = w_b[pl.ds(lax.div(t, 2) * LANES, LANES)]
                woff = lax.rem(t, 2) * K
                coefs = []
                for k in range(K):
                    v = accs[k]
                    for s in (8, 4, 2, 1):
                        v = v + v.at[lane ^ s].get(mode="promise_in_bounds")
                    e = jnp.exp(2.0 * v)
                    tanh_v = 1.0 - 2.0 / (e + 1.0)
                    wk = wvec.at[jnp.full((LANES,), woff + k, jnp.int32)].get(
                        mode="promise_in_bounds")
                    coefs.append(tanh_v * wk)

                def comb_body(c):
                    o = x_b[t, pl.ds(c, LANES)]
                    for k in range(K):
                        o = o + coefs[k] * rows_b[row0 + k, pl.ds(c, LANES)]
                    out_b[t, pl.ds(c, LANES)] = o

                plsc.parallel_loop(0, D, LANES, unroll=16)(comb_body)
                return 0

            lax.fori_loop(0, T_CHUNK, tok_body, 0)
            pltpu.async_copy(out_b, out_hbm.at[pl.ds(base_t, T_CHUNK)],
                             sem_o.at[buf])
            return 0

        lax.fori_loop(0, N_CHUNK, chunk_body, 0)

        for b in range(2):
            ci = N_CHUNK - 2 + b
            base_t = tok0 + ci * T_CHUNK
            pltpu.make_async_copy(
                out_v.at[ci % 2], out_hbm.at[pl.ds(base_t, T_CHUNK)],
                sem_o.at[ci % 2]).wait()

    return sc_kernel


def kernel(x, indices, weights, pool):
    B, D = x.shape
    K = indices.shape[1]
    idx = indices.astype(jnp.int32).reshape(-1)
    w = weights.astype(jnp.float32).reshape(-1)
    return _build(B, D, K, pool.shape[0], 4)(x, idx, w, pool)
